# baseline (speedup 1.0000x reference)
import functools

import jax
import jax.numpy as jnp
from jax import lax
from jax.experimental import pallas as pl
from jax.experimental.pallas import tpu as pltpu
from jax.experimental.pallas import tpu_sc as plsc

_CHUNK = 256


@functools.lru_cache(maxsize=None)
def _make_gather(V, D, B):
    info = plsc.get_sparse_core_info()
    NC, NS = info.num_cores, info.num_subcores
    NW = NC * NS
    assert B % NW == 0
    b_per_w = B // NW
    C = _CHUNK
    assert b_per_w % (2 * C) == 0
    n_chunks = b_per_w // C
    n_pairs = n_chunks // 2
    mesh = plsc.VectorSubcoreMesh(core_axis_name="c", subcore_axis_name="s")

    @functools.partial(
        pl.kernel,
        mesh=mesh,
        compiler_params=pltpu.CompilerParams(use_tc_tiling_on_sc=False),
        out_type=jax.ShapeDtypeStruct((B, 2 * D), jnp.float32),
        scratch_types=[
            pltpu.VMEM((b_per_w,), jnp.int32),
            pltpu.VMEM((C, 2 * D), jnp.float32),
            pltpu.VMEM((C, 2 * D), jnp.float32),
            pltpu.SemaphoreType.DMA,
            pltpu.SemaphoreType.DMA,
            pltpu.SemaphoreType.DMA,
            pltpu.SemaphoreType.DMA,
        ],
    )
    def k(table_hbm, idx_hbm, out_hbm, idx_v, rows0, rows1, g0, g1, s0, s1):
        wid = lax.axis_index("s") * NC + lax.axis_index("c")
        base = wid * b_per_w
        rows = (rows0, rows1)
        gsem = (g0, g1)
        ssem = (s0, s1)

        pltpu.sync_copy(idx_hbm.at[pl.ds(base, b_per_w)], idx_v)

        def gather(g, b):
            return pltpu.make_async_copy(
                table_hbm.at[idx_v.at[pl.ds(g * C, C)]], rows[b], gsem[b]
            )

        def store(g, b):
            return pltpu.make_async_copy(
                rows[b].at[:, pl.ds(0, D)],
                out_hbm.at[pl.ds(base + g * C, C), pl.ds(0, D)],
                ssem[b],
            )

        gather(0, 0).start()
        gather(1, 1).start()

        def body(p, carry):
            g = p * 2
            for b in (0, 1):
                gather(g + b, b).wait()
                store(g + b, b).start()
                store(g + b, b).wait()
                gather(g + b + 2, b).start()
            return carry

        lax.fori_loop(0, n_pairs - 1, body, 0)

        g = n_chunks - 2
        for b in (0, 1):
            gather(g + b, b).wait()
            store(g + b, b).start()
            store(g + b, b).wait()

    return k


def kernel(sequence, weight):
    Bs, H = sequence.shape
    V, D = weight.shape
    idx = sequence.reshape(-1).astype(jnp.int32)
    w2 = jnp.concatenate([weight, weight], axis=1)
    out = _make_gather(V, D, Bs * H)(w2, idx)
    return out[:, :D].reshape(Bs, H, D)


# speedup vs baseline: 1.1721x; 1.1721x over previous
import functools

import jax
import jax.numpy as jnp
from jax import lax
from jax.experimental import pallas as pl
from jax.experimental.pallas import tpu as pltpu
from jax.experimental.pallas import tpu_sc as plsc

_CHUNK = 512


@functools.lru_cache(maxsize=None)
def _make_gather(V, D, B):
    info = plsc.get_sparse_core_info()
    NC, NS = info.num_cores, info.num_subcores
    NW = NC * NS
    assert B % NW == 0
    b_per_w = B // NW
    C = _CHUNK
    assert b_per_w % (2 * C) == 0
    n_chunks = b_per_w // C
    n_pairs = n_chunks // 2
    mesh = plsc.VectorSubcoreMesh(core_axis_name="c", subcore_axis_name="s")

    @functools.partial(
        pl.kernel,
        mesh=mesh,
        compiler_params=pltpu.CompilerParams(use_tc_tiling_on_sc=False),
        out_type=jax.ShapeDtypeStruct((B, 2 * D), jnp.float32),
        scratch_types=[
            pltpu.VMEM((b_per_w,), jnp.int32),
            pltpu.VMEM((C, D), jnp.float32),
            pltpu.VMEM((C, D), jnp.float32),
            pltpu.SemaphoreType.DMA,
            pltpu.SemaphoreType.DMA,
            pltpu.SemaphoreType.DMA,
            pltpu.SemaphoreType.DMA,
        ],
    )
    def k(table_hbm, idx_hbm, out_hbm, idx_v, rows0, rows1, g0, g1, s0, s1):
        wid = lax.axis_index("s") * NC + lax.axis_index("c")
        base = wid * b_per_w
        rows = (rows0, rows1)
        gsem = (g0, g1)
        ssem = (s0, s1)

        pltpu.sync_copy(idx_hbm.at[pl.ds(base, b_per_w)], idx_v)

        def gather(g, b):
            return pltpu.make_async_copy(
                table_hbm.at[idx_v.at[pl.ds(g * C, C)]], rows[b], gsem[b]
            )

        def store(g, b):
            return pltpu.make_async_copy(
                rows[b], out_hbm.at[pl.ds(base + g * C, C), pl.ds(0, D)], ssem[b]
            )

        gather(0, 0).start()
        gather(1, 1).start()

        def body(p, carry):
            g = p * 2
            for b in (0, 1):
                gather(g + b, b).wait()
                store(g + b, b).start()
                store(g + b, b).wait()
                gather(g + b + 2, b).start()
            return carry

        lax.fori_loop(0, n_pairs - 1, body, 0)

        g = n_chunks - 2
        for b in (0, 1):
            gather(g + b, b).wait()
            store(g + b, b).start()
            store(g + b, b).wait()

    return k


def kernel(sequence, weight):
    Bs, H = sequence.shape
    V, D = weight.shape
    idx = sequence.reshape(-1).astype(jnp.int32)
    out = _make_gather(V, D, Bs * H)(weight, idx)
    return out[:, :D].reshape(Bs, H, D)
